# baseline (speedup 1.0000x reference)
import functools

import jax
import jax.numpy as jnp
from jax import lax
from jax.experimental import pallas as pl
from jax.experimental.pallas import tpu as pltpu
from jax.experimental.pallas import tpu_sc as plsc

EMB_DIM = 128
BATCH = 16384
NUM_CORES = 2
NUM_SUBCORES = 16
NUM_WORKERS = NUM_CORES * NUM_SUBCORES
B_PER_W = BATCH // NUM_WORKERS
CHUNK = 64
N_CHUNKS = B_PER_W // CHUNK

_mesh = plsc.VectorSubcoreMesh(core_axis_name="c", subcore_axis_name="s")


@functools.partial(
    pl.kernel,
    mesh=_mesh,
    out_type=jax.ShapeDtypeStruct((BATCH, EMB_DIM), jnp.float32),
    scratch_types=[
        pltpu.VMEM((N_CHUNKS, CHUNK), jnp.int32),
        pltpu.VMEM((B_PER_W, EMB_DIM), jnp.float32),
    ]
    + [pltpu.SemaphoreType.DMA] * N_CHUNKS
    + [pltpu.SemaphoreType.DMA],
)
def _gather_kernel(table_hbm, idx_hbm, out_hbm, idx_v, rows_v, *sems):
    gsems, wsem = sems[:N_CHUNKS], sems[N_CHUNKS]
    wid = lax.axis_index("s") * NUM_CORES + lax.axis_index("c")
    base = wid * B_PER_W
    pltpu.sync_copy(idx_hbm.at[wid], idx_v)
    gathers = [
        pltpu.async_copy(
            table_hbm.at[idx_v.at[j]],
            rows_v.at[pl.ds(j * CHUNK, CHUNK)],
            gsems[j],
        )
        for j in range(N_CHUNKS)
    ]
    writes = []
    for j in range(N_CHUNKS):
        gathers[j].wait()
        writes.append(
            pltpu.async_copy(
                rows_v.at[pl.ds(j * CHUNK, CHUNK)],
                out_hbm.at[pl.ds(base + j * CHUNK, CHUNK)],
                wsem,
            )
        )
    for w in writes:
        w.wait()


def kernel(idx, table):
    idx3 = idx.astype(jnp.int32).reshape(NUM_WORKERS, N_CHUNKS, CHUNK)
    out = _gather_kernel(table, idx3)
    return out.reshape(-1, EMB_DIM, 1, 1)


# speedup vs baseline: 1.0393x; 1.0393x over previous
import functools

import jax
import jax.numpy as jnp
from jax import lax
from jax.experimental import pallas as pl
from jax.experimental.pallas import tpu as pltpu
from jax.experimental.pallas import tpu_sc as plsc

EMB_DIM = 128
BATCH = 16384
NUM_CORES = 2
NUM_SUBCORES = 16
NUM_WORKERS = NUM_CORES * NUM_SUBCORES
B_PER_W = BATCH // NUM_WORKERS
CHUNK = 512
N_CHUNKS = B_PER_W // CHUNK

_mesh = plsc.VectorSubcoreMesh(core_axis_name="c", subcore_axis_name="s")


@functools.partial(
    pl.kernel,
    mesh=_mesh,
    out_type=jax.ShapeDtypeStruct((BATCH, EMB_DIM), jnp.float32),
    scratch_types=[
        pltpu.VMEM((N_CHUNKS, CHUNK), jnp.int32),
        pltpu.VMEM((B_PER_W, EMB_DIM), jnp.float32),
    ]
    + [pltpu.SemaphoreType.DMA] * N_CHUNKS
    + [pltpu.SemaphoreType.DMA],
)
def _gather_kernel(table_hbm, idx_hbm, out_hbm, idx_v, rows_v, *sems):
    gsems, wsem = sems[:N_CHUNKS], sems[N_CHUNKS]
    wid = lax.axis_index("s") * NUM_CORES + lax.axis_index("c")
    base = wid * B_PER_W
    pltpu.sync_copy(idx_hbm.at[wid], idx_v)
    gathers = [
        pltpu.async_copy(
            table_hbm.at[idx_v.at[j]],
            rows_v.at[pl.ds(j * CHUNK, CHUNK)],
            gsems[j],
        )
        for j in range(N_CHUNKS)
    ]
    writes = []
    for j in range(N_CHUNKS):
        gathers[j].wait()
        writes.append(
            pltpu.async_copy(
                rows_v.at[pl.ds(j * CHUNK, CHUNK)],
                out_hbm.at[pl.ds(base + j * CHUNK, CHUNK)],
                wsem,
            )
        )
    for w in writes:
        w.wait()


def kernel(idx, table):
    idx3 = idx.astype(jnp.int32).reshape(NUM_WORKERS, N_CHUNKS, CHUNK)
    out = _gather_kernel(table, idx3)
    return out.reshape(-1, EMB_DIM, 1, 1)
